# baseline (speedup 1.0000x reference)
import functools

import jax
import jax.numpy as jnp
from jax import lax
from jax.experimental import pallas as pl
from jax.experimental.pallas import tpu as pltpu
from jax.experimental.pallas import tpu_sc as plsc

N_NODES = 10000
N_EDGES = 320000
N_GRAPHS = 64
CHUNK = 128
NSUB = 16
NCORES = 2
E_PAD = 327680
NCHUNKS = E_PAD // CHUNK
ACC_ROWS = N_NODES + 112
RPS = ACC_ROWS // NSUB
HALF = 64
IDX_BLK = 16


@functools.lru_cache(maxsize=None)
def _make_seg_sum(edge_split: bool):
    nchunks_w = NCHUNKS // (NSUB * (2 if edge_split else 1))
    ngroups = nchunks_w // IDX_BLK
    mesh = plsc.VectorSubcoreMesh(core_axis_name="c", subcore_axis_name="s")

    @functools.partial(
        pl.kernel,
        out_type=jax.ShapeDtypeStruct((NCORES, ACC_ROWS, HALF), jnp.float32),
        mesh=mesh,
        scratch_types=[
            pltpu.VMEM_SHARED((ACC_ROWS, HALF), jnp.float32),
            pltpu.VMEM_SHARED((N_NODES, HALF), jnp.float32),
            pltpu.VMEM((IDX_BLK, CHUNK), jnp.int32),
            pltpu.VMEM((IDX_BLK, CHUNK), jnp.int32),
            pltpu.VMEM((CHUNK, HALF), jnp.float32),
            pltpu.VMEM((CHUNK, HALF), jnp.float32),
            pltpu.SemaphoreType.DMA,
            pltpu.SemaphoreType.DMA,
        ],
        compiler_params=pltpu.CompilerParams(use_tc_tiling_on_sc=False),
    )
    def seg_sum(op0, op1, src2d, dst2d, zeros, out, acc, op_sh, src_v, dst_v,
                rows_a, rows_b, sem_a, sem_b):
        c = lax.axis_index("c")
        s = lax.axis_index("s")
        row_base = pl.multiple_of(s * RPS, 8)
        pltpu.sync_copy(zeros, acc.at[pl.ds(row_base, RPS)])

        if edge_split:
            chunk_base = c * (NCHUNKS // 2) + s * nchunks_w
        else:
            chunk_base = s * nchunks_w
        chunk_base = pl.multiple_of(chunk_base, 8)

        def stage(op_ref):
            @pl.when(s < NSUB - 1)
            def _():
                b = pl.multiple_of(s * RPS, 8)
                pltpu.sync_copy(op_ref.at[pl.ds(b, RPS)],
                                op_sh.at[pl.ds(b, RPS)])

            @pl.when(s == NSUB - 1)
            def _():
                tail = N_NODES - (NSUB - 1) * RPS
                pltpu.sync_copy(op_ref.at[pl.ds((NSUB - 1) * RPS, tail)],
                                op_sh.at[pl.ds((NSUB - 1) * RPS, tail)])

        @pl.when(c == 0)
        def _():
            stage(op0)

        @pl.when(c == 1)
        def _():
            stage(op1)

        plsc.subcore_barrier()

        def run():
            bufs = (rows_a, rows_b)
            sems = (sem_a, sem_b)

            def group(gi, carry):
                gbase = pl.multiple_of(chunk_base + gi * IDX_BLK, 8)
                pltpu.sync_copy(src2d.at[pl.ds(gbase, IDX_BLK)], src_v)
                pltpu.sync_copy(dst2d.at[pl.ds(gbase, IDX_BLK)], dst_v)
                pltpu.async_copy(op_sh.at[src_v.at[0]], rows_a, sem_a)

                def body(p, carry2):
                    for b in range(2):
                        j = 2 * p + b
                        nxt = j + 1

                        @pl.when(nxt < IDX_BLK)
                        def _():
                            pltpu.async_copy(op_sh.at[src_v.at[nxt]],
                                             bufs[1 - b], sems[1 - b])

                        pltpu.make_async_copy(op_sh.at[src_v.at[j]],
                                              bufs[b], sems[b]).wait()
                        pltpu.sync_copy(bufs[b], acc.at[dst_v.at[j]],
                                        add=True)
                    return carry2

                lax.fori_loop(0, IDX_BLK // 2, body, 0)
                return carry

            lax.fori_loop(0, ngroups, group, 0)

        run()

        plsc.subcore_barrier()
        pltpu.sync_copy(acc.at[pl.ds(row_base, RPS)],
                        out.at[c, pl.ds(row_base, RPS)])

    return seg_sum


def _seg_sum_feat(*args):
    return _make_seg_sum(edge_split=False)(*args)


def _seg_sum_edge(*args):
    return _make_seg_sum(edge_split=True)(*args)


def _elu(v):
    return jnp.where(v > 0, v, jnp.exp(jnp.minimum(v, 0.0)) - 1.0)


ROWS_BLK = 1000
N_BLKS = N_NODES // ROWS_BLK


def _mlp1_kernel(x_ref, agg_ref, W1_ref, b1_ref, g_ref, be_ref, mu_ref,
                 var_ref, W2_ref, b2_ref, out_ref):
    x = x_ref[...]
    a = agg_ref[...]
    z0 = x[:, :HALF] + a[0]
    z1 = x[:, HALF:] + a[1]
    W1 = W1_ref[...]
    h = (jnp.dot(z0, W1[:HALF], preferred_element_type=jnp.float32)
         + jnp.dot(z1, W1[HALF:], preferred_element_type=jnp.float32)
         + b1_ref[...])
    scale = g_ref[...] * lax.rsqrt(var_ref[...] + 1e-5)
    h = (h - mu_ref[...]) * scale + be_ref[...]
    h = _elu(h)
    h = jnp.dot(h, W2_ref[...], preferred_element_type=jnp.float32) + b2_ref[...]
    out_ref[...] = _elu(h)


def _mlp2_kernel(h_ref, q_ref, b3d_ref, W3_ref, b3_ref, W4_ref, b4_ref,
                 Wr1_ref, br1_ref, Wr2_ref, br2_ref, Wr3_ref, br3_ref,
                 out_ref, g_acc):
    i = pl.program_id(0)
    h = h_ref[...]
    q = q_ref[...]
    z = h + q[0] + q[1]
    t = _elu(jnp.dot(z, W3_ref[...], preferred_element_type=jnp.float32)
             + b3_ref[...])
    p = _elu(jnp.dot(t, W4_ref[...], preferred_element_type=jnp.float32)
             + b4_ref[...])
    bvals = b3d_ref[0]
    onehot_t = (lax.broadcasted_iota(jnp.int32, (N_GRAPHS, ROWS_BLK), 0)
                == bvals).astype(jnp.float32)
    contrib = jnp.dot(onehot_t, p, preferred_element_type=jnp.float32)

    @pl.when(i == 0)
    def _():
        g_acc[...] = contrib

    @pl.when(i > 0)
    def _():
        g_acc[...] = g_acc[...] + contrib

    @pl.when(i == N_BLKS - 1)
    def _():
        g = g_acc[...]
        r = _elu(jnp.dot(g, Wr1_ref[...], preferred_element_type=jnp.float32)
                 + br1_ref[...])
        r = _elu(jnp.dot(r, Wr2_ref[...], preferred_element_type=jnp.float32)
                 + br2_ref[...])
        out_ref[...] = (jnp.dot(r, Wr3_ref[...],
                                preferred_element_type=jnp.float32)
                        + br3_ref[...])


def _full_spec(shape):
    return pl.BlockSpec(shape, lambda i, _s=shape: tuple(0 for _ in _s))


def _mlp1(x, agg, W1, b1, gamma, beta, mean, var, W2, b2):
    return pl.pallas_call(
        _mlp1_kernel,
        grid=(N_BLKS,),
        in_specs=[
            pl.BlockSpec((ROWS_BLK, 128), lambda i: (i, 0)),
            pl.BlockSpec((NCORES, ROWS_BLK, HALF), lambda i: (0, i, 0)),
            _full_spec((128, HALF)),
            _full_spec((1, HALF)),
            _full_spec((1, HALF)),
            _full_spec((1, HALF)),
            _full_spec((1, HALF)),
            _full_spec((1, HALF)),
            _full_spec((HALF, HALF)),
            _full_spec((1, HALF)),
        ],
        out_specs=pl.BlockSpec((ROWS_BLK, HALF), lambda i: (i, 0)),
        out_shape=jax.ShapeDtypeStruct((N_NODES, HALF), jnp.float32),
    )(x, agg, W1, b1.reshape(1, -1), gamma.reshape(1, -1),
      beta.reshape(1, -1), mean.reshape(1, -1), var.reshape(1, -1), W2,
      b2.reshape(1, -1))


def _mlp2(h, q, batch3d, W3, b3, W4, b4, Wr1, br1, Wr2, br2, Wr3, br3):
    return pl.pallas_call(
        _mlp2_kernel,
        grid=(N_BLKS,),
        in_specs=[
            pl.BlockSpec((ROWS_BLK, HALF), lambda i: (i, 0)),
            pl.BlockSpec((NCORES, ROWS_BLK, HALF), lambda i: (0, i, 0)),
            pl.BlockSpec((1, 1, ROWS_BLK), lambda i: (i, 0, 0)),
            _full_spec((HALF, HALF)),
            _full_spec((1, HALF)),
            _full_spec((HALF, HALF)),
            _full_spec((1, HALF)),
            _full_spec((HALF, HALF)),
            _full_spec((1, HALF)),
            _full_spec((HALF, 32)),
            _full_spec((1, 32)),
            _full_spec((32, 10)),
            _full_spec((1, 10)),
        ],
        out_specs=_full_spec((N_GRAPHS, 10)),
        out_shape=jax.ShapeDtypeStruct((N_GRAPHS, 10), jnp.float32),
        scratch_shapes=[pltpu.VMEM((N_GRAPHS, N_GRAPHS), jnp.float32)],
    )(h, q, batch3d, W3, b3.reshape(1, -1), W4, b4.reshape(1, -1),
      Wr1, br1.reshape(1, -1), Wr2, br2.reshape(1, -1), Wr3,
      br3.reshape(1, -1))


def kernel(x, edge_index, batch, W1, b1, gamma, beta, mean, var, W2, b2,
           W3, b3, W4, b4, Wr1, br1, Wr2, br2, Wr3, br3):
    src, dst = edge_index[0], edge_index[1]
    pad = E_PAD - N_EDGES
    pad_idx = jnp.arange(pad, dtype=jnp.int32)
    src_p = jnp.concatenate([src, (pad_idx * 97) % N_NODES])
    dst_p = jnp.concatenate([dst, N_NODES + (pad_idx % 16)])
    src2d = src_p.reshape(NCHUNKS, CHUNK)
    dst2d = dst_p.reshape(NCHUNKS, CHUNK)
    zeros = jnp.zeros((RPS, HALF), jnp.float32)

    xa = x[:, :HALF]
    xb = x[:, HALF:]
    agg1 = _seg_sum_feat(xa, xb, src2d, dst2d, zeros)
    h = _mlp1(x, agg1, W1, b1, gamma, beta, mean, var, W2, b2)
    agg2 = _seg_sum_edge(h, h, src2d, dst2d, zeros)
    batch3d = batch.reshape(N_BLKS, 1, ROWS_BLK)
    logits = _mlp2(h, agg2, batch3d, W3, b3, W4, b4, Wr1, br1, Wr2, br2,
                   Wr3, br3)
    return (logits, jnp.zeros((), jnp.float32))


# speedup vs baseline: 1.2431x; 1.2431x over previous
import functools

import jax
import jax.numpy as jnp
from jax import lax
from jax.experimental import pallas as pl
from jax.experimental.pallas import tpu as pltpu
from jax.experimental.pallas import tpu_sc as plsc

N_NODES = 10000
N_EDGES = 320000
N_GRAPHS = 64
CHUNK = 128
NSUB = 16
NCORES = 2
E_PAD = 327680
NCHUNKS = E_PAD // CHUNK
ACC_ROWS = N_NODES + 112
RPS = ACC_ROWS // NSUB
HALF = 64


@functools.lru_cache(maxsize=None)
def _make_seg_sum(edge_split: bool):
    nchunks_w = NCHUNKS // (NSUB * (2 if edge_split else 1))
    mesh = plsc.VectorSubcoreMesh(core_axis_name="c", subcore_axis_name="s")

    @functools.partial(
        pl.kernel,
        out_type=jax.ShapeDtypeStruct((NCORES, ACC_ROWS, HALF), jnp.float32),
        mesh=mesh,
        scratch_types=[
            pltpu.VMEM_SHARED((ACC_ROWS, HALF), jnp.float32),
            pltpu.VMEM((nchunks_w, CHUNK), jnp.int32),
            pltpu.VMEM((nchunks_w, CHUNK), jnp.int32),
            pltpu.VMEM((CHUNK, HALF), jnp.float32),
            pltpu.VMEM((CHUNK, HALF), jnp.float32),
            pltpu.SemaphoreType.DMA,
            pltpu.SemaphoreType.DMA,
        ],
        compiler_params=pltpu.CompilerParams(use_tc_tiling_on_sc=False),
    )
    def seg_sum(op0, op1, src2d, dst2d, zeros, out, acc, src_v, dst_v,
                rows_a, rows_b, sem_a, sem_b):
        c = lax.axis_index("c")
        s = lax.axis_index("s")
        row_base = pl.multiple_of(s * RPS, 8)
        pltpu.sync_copy(zeros, acc.at[pl.ds(row_base, RPS)])

        if edge_split:
            chunk_base = c * (NCHUNKS // 2) + s * nchunks_w
        else:
            chunk_base = s * nchunks_w
        chunk_base = pl.multiple_of(chunk_base, 8)
        pltpu.sync_copy(src2d.at[pl.ds(chunk_base, nchunks_w)], src_v)
        pltpu.sync_copy(dst2d.at[pl.ds(chunk_base, nchunks_w)], dst_v)
        plsc.subcore_barrier()

        def run(op_ref):
            bufs = (rows_a, rows_b)
            sems = (sem_a, sem_b)
            pltpu.async_copy(op_ref.at[src_v.at[0]], rows_a, sem_a)

            def body(p, carry):
                for b in range(2):
                    j = 2 * p + b
                    nxt = j + 1

                    @pl.when(nxt < nchunks_w)
                    def _():
                        pltpu.async_copy(op_ref.at[src_v.at[nxt]],
                                         bufs[1 - b], sems[1 - b])

                    pltpu.make_async_copy(op_ref.at[src_v.at[j]],
                                          bufs[b], sems[b]).wait()
                    pltpu.sync_copy(bufs[b], acc.at[dst_v.at[j]], add=True)
                return carry

            lax.fori_loop(0, nchunks_w // 2, body, 0)

        @pl.when(c == 0)
        def _():
            run(op0)

        @pl.when(c == 1)
        def _():
            run(op1)

        plsc.subcore_barrier()
        pltpu.sync_copy(acc.at[pl.ds(row_base, RPS)],
                        out.at[c, pl.ds(row_base, RPS)])

    return seg_sum


def _seg_sum_feat(*args):
    return _make_seg_sum(edge_split=False)(*args)


def _seg_sum_edge(*args):
    return _make_seg_sum(edge_split=True)(*args)


def _elu(v):
    return jnp.where(v > 0, v, jnp.exp(jnp.minimum(v, 0.0)) - 1.0)


ROWS_BLK = 1000
N_BLKS = N_NODES // ROWS_BLK


def _mlp1_kernel(x_ref, agg_ref, W1_ref, b1_ref, g_ref, be_ref, mu_ref,
                 var_ref, W2_ref, b2_ref, out_ref):
    x = x_ref[...]
    a = agg_ref[...]
    z0 = x[:, :HALF] + a[0]
    z1 = x[:, HALF:] + a[1]
    W1 = W1_ref[...]
    h = (jnp.dot(z0, W1[:HALF], preferred_element_type=jnp.float32)
         + jnp.dot(z1, W1[HALF:], preferred_element_type=jnp.float32)
         + b1_ref[...])
    scale = g_ref[...] * lax.rsqrt(var_ref[...] + 1e-5)
    h = (h - mu_ref[...]) * scale + be_ref[...]
    h = _elu(h)
    h = jnp.dot(h, W2_ref[...], preferred_element_type=jnp.float32) + b2_ref[...]
    out_ref[...] = _elu(h)


def _mlp2_kernel(h_ref, q_ref, b3d_ref, W3_ref, b3_ref, W4_ref, b4_ref,
                 Wr1_ref, br1_ref, Wr2_ref, br2_ref, Wr3_ref, br3_ref,
                 out_ref, g_acc):
    i = pl.program_id(0)
    h = h_ref[...]
    q = q_ref[...]
    z = h + q[0] + q[1]
    t = _elu(jnp.dot(z, W3_ref[...], preferred_element_type=jnp.float32)
             + b3_ref[...])
    p = _elu(jnp.dot(t, W4_ref[...], preferred_element_type=jnp.float32)
             + b4_ref[...])
    bvals = b3d_ref[0]
    onehot_t = (lax.broadcasted_iota(jnp.int32, (N_GRAPHS, ROWS_BLK), 0)
                == bvals).astype(jnp.float32)
    contrib = jnp.dot(onehot_t, p, preferred_element_type=jnp.float32)

    @pl.when(i == 0)
    def _():
        g_acc[...] = contrib

    @pl.when(i > 0)
    def _():
        g_acc[...] = g_acc[...] + contrib

    @pl.when(i == N_BLKS - 1)
    def _():
        g = g_acc[...]
        r = _elu(jnp.dot(g, Wr1_ref[...], preferred_element_type=jnp.float32)
                 + br1_ref[...])
        r = _elu(jnp.dot(r, Wr2_ref[...], preferred_element_type=jnp.float32)
                 + br2_ref[...])
        out_ref[...] = (jnp.dot(r, Wr3_ref[...],
                                preferred_element_type=jnp.float32)
                        + br3_ref[...])


def _full_spec(shape):
    return pl.BlockSpec(shape, lambda i, _s=shape: tuple(0 for _ in _s))


def _mlp1(x, agg, W1, b1, gamma, beta, mean, var, W2, b2):
    return pl.pallas_call(
        _mlp1_kernel,
        grid=(N_BLKS,),
        in_specs=[
            pl.BlockSpec((ROWS_BLK, 128), lambda i: (i, 0)),
            pl.BlockSpec((NCORES, ROWS_BLK, HALF), lambda i: (0, i, 0)),
            _full_spec((128, HALF)),
            _full_spec((1, HALF)),
            _full_spec((1, HALF)),
            _full_spec((1, HALF)),
            _full_spec((1, HALF)),
            _full_spec((1, HALF)),
            _full_spec((HALF, HALF)),
            _full_spec((1, HALF)),
        ],
        out_specs=pl.BlockSpec((ROWS_BLK, HALF), lambda i: (i, 0)),
        out_shape=jax.ShapeDtypeStruct((N_NODES, HALF), jnp.float32),
    )(x, agg, W1, b1.reshape(1, -1), gamma.reshape(1, -1),
      beta.reshape(1, -1), mean.reshape(1, -1), var.reshape(1, -1), W2,
      b2.reshape(1, -1))


def _mlp2(h, q, batch3d, W3, b3, W4, b4, Wr1, br1, Wr2, br2, Wr3, br3):
    return pl.pallas_call(
        _mlp2_kernel,
        grid=(N_BLKS,),
        in_specs=[
            pl.BlockSpec((ROWS_BLK, HALF), lambda i: (i, 0)),
            pl.BlockSpec((NCORES, ROWS_BLK, HALF), lambda i: (0, i, 0)),
            pl.BlockSpec((1, 1, ROWS_BLK), lambda i: (i, 0, 0)),
            _full_spec((HALF, HALF)),
            _full_spec((1, HALF)),
            _full_spec((HALF, HALF)),
            _full_spec((1, HALF)),
            _full_spec((HALF, HALF)),
            _full_spec((1, HALF)),
            _full_spec((HALF, 32)),
            _full_spec((1, 32)),
            _full_spec((32, 10)),
            _full_spec((1, 10)),
        ],
        out_specs=_full_spec((N_GRAPHS, 10)),
        out_shape=jax.ShapeDtypeStruct((N_GRAPHS, 10), jnp.float32),
        scratch_shapes=[pltpu.VMEM((N_GRAPHS, N_GRAPHS), jnp.float32)],
    )(h, q, batch3d, W3, b3.reshape(1, -1), W4, b4.reshape(1, -1),
      Wr1, br1.reshape(1, -1), Wr2, br2.reshape(1, -1), Wr3,
      br3.reshape(1, -1))


def kernel(x, edge_index, batch, W1, b1, gamma, beta, mean, var, W2, b2,
           W3, b3, W4, b4, Wr1, br1, Wr2, br2, Wr3, br3):
    src, dst = edge_index[0], edge_index[1]
    pad = E_PAD - N_EDGES
    pad_idx = jnp.arange(pad, dtype=jnp.int32)
    src_p = jnp.concatenate([src, (pad_idx * 97) % N_NODES])
    dst_p = jnp.concatenate([dst, N_NODES + (pad_idx % 16)])
    src2d = src_p.reshape(NCHUNKS, CHUNK)
    dst2d = dst_p.reshape(NCHUNKS, CHUNK)
    zeros = jnp.zeros((RPS, HALF), jnp.float32)

    xa = x[:, :HALF]
    xb = x[:, HALF:]
    agg1 = _seg_sum_feat(xa, xb, src2d, dst2d, zeros)
    h = _mlp1(x, agg1, W1, b1, gamma, beta, mean, var, W2, b2)
    agg2 = _seg_sum_edge(h, h, src2d, dst2d, zeros)
    batch3d = batch.reshape(N_BLKS, 1, ROWS_BLK)
    logits = _mlp2(h, agg2, batch3d, W3, b3, W4, b4, Wr1, br1, Wr2, br2,
                   Wr3, br3)
    return (logits, jnp.zeros((), jnp.float32))


# speedup vs baseline: 1.3937x; 1.1212x over previous
import functools

import jax
import jax.numpy as jnp
from jax import lax
from jax.experimental import pallas as pl
from jax.experimental.pallas import tpu as pltpu
from jax.experimental.pallas import tpu_sc as plsc

N_NODES = 10000
N_EDGES = 320000
N_GRAPHS = 64
CHUNK = 128
NSUB = 16
NCORES = 2
E_PAD = 327680
NCHUNKS = E_PAD // CHUNK
ACC_ROWS = N_NODES + 112
RPS = ACC_ROWS // NSUB
HALF = 64


@functools.lru_cache(maxsize=None)
def _make_seg_sum(edge_split: bool):
    nchunks_w = NCHUNKS // (NSUB * (2 if edge_split else 1))
    mesh = plsc.VectorSubcoreMesh(core_axis_name="c", subcore_axis_name="s")

    @functools.partial(
        pl.kernel,
        out_type=jax.ShapeDtypeStruct((NCORES, ACC_ROWS, HALF), jnp.float32),
        mesh=mesh,
        scratch_types=[
            pltpu.VMEM_SHARED((ACC_ROWS, HALF), jnp.float32),
            pltpu.VMEM((nchunks_w, CHUNK), jnp.int32),
            pltpu.VMEM((nchunks_w, CHUNK), jnp.int32),
            [pltpu.VMEM((CHUNK, HALF), jnp.float32)] * 4,
            [pltpu.SemaphoreType.DMA] * 4,
            [pltpu.SemaphoreType.DMA] * 4,
        ],
        compiler_params=pltpu.CompilerParams(use_tc_tiling_on_sc=False),
    )
    def seg_sum(op0, op1, src2d, dst2d, zeros, out, acc, src_v, dst_v,
                bufs, gsem, ssem):
        c = lax.axis_index("c")
        s = lax.axis_index("s")
        row_base = pl.multiple_of(s * RPS, 8)
        pltpu.sync_copy(zeros, acc.at[pl.ds(row_base, RPS)])

        if edge_split:
            chunk_base = c * (NCHUNKS // 2) + s * nchunks_w
        else:
            chunk_base = s * nchunks_w
        chunk_base = pl.multiple_of(chunk_base, 8)
        pltpu.sync_copy(src2d.at[pl.ds(chunk_base, nchunks_w)], src_v)
        pltpu.sync_copy(dst2d.at[pl.ds(chunk_base, nchunks_w)], dst_v)
        plsc.subcore_barrier()

        def run(op_ref):
            n = nchunks_w

            def gather(j, b):
                pltpu.async_copy(op_ref.at[src_v.at[j]], bufs[b], gsem[b])

            def wait_gather(j, b):
                pltpu.make_async_copy(op_ref.at[src_v.at[j]], bufs[b],
                                      gsem[b]).wait()

            def scatter(j, b):
                pltpu.async_copy(bufs[b], acc.at[dst_v.at[j]], ssem[b],
                                 add=True)

            def wait_scatter(j, b):
                pltpu.make_async_copy(bufs[b], acc.at[dst_v.at[j]],
                                      ssem[b]).wait()

            for b in range(3):
                gather(b, b)
            for b in range(4):
                wait_gather(b, b)
                scatter(b, b)
                if b == 0:
                    gather(3, 3)
                else:
                    wait_scatter(b - 1, (b + 3) % 4)
                    gather(b + 3, (b + 3) % 4)

            def body(p, carry):
                for b in range(4):
                    j = 4 * p + b
                    wait_gather(j, b)
                    scatter(j, b)
                    wait_scatter(j - 1, (b + 3) % 4)
                    gather(j + 3, (b + 3) % 4)
                return carry

            lax.fori_loop(1, n // 4 - 1, body, 0)

            for b in range(4):
                j = n - 4 + b
                if b == 0:
                    wait_scatter(j - 1, 3)
                    gather(j + 3, 3)
                wait_gather(j, b)
                pltpu.sync_copy(bufs[b], acc.at[dst_v.at[j]], add=True)

        @pl.when(c == 0)
        def _():
            run(op0)

        @pl.when(c == 1)
        def _():
            run(op1)

        plsc.subcore_barrier()
        pltpu.sync_copy(acc.at[pl.ds(row_base, RPS)],
                        out.at[c, pl.ds(row_base, RPS)])

    return seg_sum


def _seg_sum_feat(*args):
    return _make_seg_sum(edge_split=False)(*args)


def _seg_sum_edge(*args):
    return _make_seg_sum(edge_split=True)(*args)


def _elu(v):
    return jnp.where(v > 0, v, jnp.exp(jnp.minimum(v, 0.0)) - 1.0)


ROWS_BLK = 1000
N_BLKS = N_NODES // ROWS_BLK


def _mlp1_kernel(x_ref, agg_ref, W1_ref, b1_ref, g_ref, be_ref, mu_ref,
                 var_ref, W2_ref, b2_ref, out_ref):
    x = x_ref[...]
    a = agg_ref[...]
    z0 = x[:, :HALF] + a[0]
    z1 = x[:, HALF:] + a[1]
    W1 = W1_ref[...]
    h = (jnp.dot(z0, W1[:HALF], preferred_element_type=jnp.float32)
         + jnp.dot(z1, W1[HALF:], preferred_element_type=jnp.float32)
         + b1_ref[...])
    scale = g_ref[...] * lax.rsqrt(var_ref[...] + 1e-5)
    h = (h - mu_ref[...]) * scale + be_ref[...]
    h = _elu(h)
    h = jnp.dot(h, W2_ref[...], preferred_element_type=jnp.float32) + b2_ref[...]
    out_ref[...] = _elu(h)


def _mlp2_kernel(h_ref, q_ref, b3d_ref, W3_ref, b3_ref, W4_ref, b4_ref,
                 Wr1_ref, br1_ref, Wr2_ref, br2_ref, Wr3_ref, br3_ref,
                 out_ref, g_acc):
    i = pl.program_id(0)
    h = h_ref[...]
    q = q_ref[...]
    z = h + q[0] + q[1]
    t = _elu(jnp.dot(z, W3_ref[...], preferred_element_type=jnp.float32)
             + b3_ref[...])
    p = _elu(jnp.dot(t, W4_ref[...], preferred_element_type=jnp.float32)
             + b4_ref[...])
    bvals = b3d_ref[0]
    onehot_t = (lax.broadcasted_iota(jnp.int32, (N_GRAPHS, ROWS_BLK), 0)
                == bvals).astype(jnp.float32)
    contrib = jnp.dot(onehot_t, p, preferred_element_type=jnp.float32)

    @pl.when(i == 0)
    def _():
        g_acc[...] = contrib

    @pl.when(i > 0)
    def _():
        g_acc[...] = g_acc[...] + contrib

    @pl.when(i == N_BLKS - 1)
    def _():
        g = g_acc[...]
        r = _elu(jnp.dot(g, Wr1_ref[...], preferred_element_type=jnp.float32)
                 + br1_ref[...])
        r = _elu(jnp.dot(r, Wr2_ref[...], preferred_element_type=jnp.float32)
                 + br2_ref[...])
        out_ref[...] = (jnp.dot(r, Wr3_ref[...],
                                preferred_element_type=jnp.float32)
                        + br3_ref[...])


def _full_spec(shape):
    return pl.BlockSpec(shape, lambda i, _s=shape: tuple(0 for _ in _s))


def _mlp1(x, agg, W1, b1, gamma, beta, mean, var, W2, b2):
    return pl.pallas_call(
        _mlp1_kernel,
        grid=(N_BLKS,),
        in_specs=[
            pl.BlockSpec((ROWS_BLK, 128), lambda i: (i, 0)),
            pl.BlockSpec((NCORES, ROWS_BLK, HALF), lambda i: (0, i, 0)),
            _full_spec((128, HALF)),
            _full_spec((1, HALF)),
            _full_spec((1, HALF)),
            _full_spec((1, HALF)),
            _full_spec((1, HALF)),
            _full_spec((1, HALF)),
            _full_spec((HALF, HALF)),
            _full_spec((1, HALF)),
        ],
        out_specs=pl.BlockSpec((ROWS_BLK, HALF), lambda i: (i, 0)),
        out_shape=jax.ShapeDtypeStruct((N_NODES, HALF), jnp.float32),
    )(x, agg, W1, b1.reshape(1, -1), gamma.reshape(1, -1),
      beta.reshape(1, -1), mean.reshape(1, -1), var.reshape(1, -1), W2,
      b2.reshape(1, -1))


def _mlp2(h, q, batch3d, W3, b3, W4, b4, Wr1, br1, Wr2, br2, Wr3, br3):
    return pl.pallas_call(
        _mlp2_kernel,
        grid=(N_BLKS,),
        in_specs=[
            pl.BlockSpec((ROWS_BLK, HALF), lambda i: (i, 0)),
            pl.BlockSpec((NCORES, ROWS_BLK, HALF), lambda i: (0, i, 0)),
            pl.BlockSpec((1, 1, ROWS_BLK), lambda i: (i, 0, 0)),
            _full_spec((HALF, HALF)),
            _full_spec((1, HALF)),
            _full_spec((HALF, HALF)),
            _full_spec((1, HALF)),
            _full_spec((HALF, HALF)),
            _full_spec((1, HALF)),
            _full_spec((HALF, 32)),
            _full_spec((1, 32)),
            _full_spec((32, 10)),
            _full_spec((1, 10)),
        ],
        out_specs=_full_spec((N_GRAPHS, 10)),
        out_shape=jax.ShapeDtypeStruct((N_GRAPHS, 10), jnp.float32),
        scratch_shapes=[pltpu.VMEM((N_GRAPHS, N_GRAPHS), jnp.float32)],
    )(h, q, batch3d, W3, b3.reshape(1, -1), W4, b4.reshape(1, -1),
      Wr1, br1.reshape(1, -1), Wr2, br2.reshape(1, -1), Wr3,
      br3.reshape(1, -1))


def kernel(x, edge_index, batch, W1, b1, gamma, beta, mean, var, W2, b2,
           W3, b3, W4, b4, Wr1, br1, Wr2, br2, Wr3, br3):
    src, dst = edge_index[0], edge_index[1]
    pad = E_PAD - N_EDGES
    pad_idx = jnp.arange(pad, dtype=jnp.int32)
    src_p = jnp.concatenate([src, (pad_idx * 97) % N_NODES])
    dst_p = jnp.concatenate([dst, N_NODES + (pad_idx % 16)])
    src2d = src_p.reshape(NCHUNKS, CHUNK)
    dst2d = dst_p.reshape(NCHUNKS, CHUNK)
    zeros = jnp.zeros((RPS, HALF), jnp.float32)

    xa = x[:, :HALF]
    xb = x[:, HALF:]
    agg1 = _seg_sum_feat(xa, xb, src2d, dst2d, zeros)
    h = _mlp1(x, agg1, W1, b1, gamma, beta, mean, var, W2, b2)
    agg2 = _seg_sum_edge(h, h, src2d, dst2d, zeros)
    batch3d = batch.reshape(N_BLKS, 1, ROWS_BLK)
    logits = _mlp2(h, agg2, batch3d, W3, b3, W4, b4, Wr1, br1, Wr2, br2,
                   Wr3, br3)
    return (logits, jnp.zeros((), jnp.float32))


# speedup vs baseline: 1.4143x; 1.0148x over previous
import functools

import jax
import jax.numpy as jnp
from jax import lax
from jax.experimental import pallas as pl
from jax.experimental.pallas import tpu as pltpu
from jax.experimental.pallas import tpu_sc as plsc

N_NODES = 10000
N_EDGES = 320000
N_GRAPHS = 64
CHUNK = 128
NSUB = 16
NCORES = 2
E_PAD = 327680
NCHUNKS = E_PAD // CHUNK
ACC_ROWS = N_NODES + 112
RPS = ACC_ROWS // NSUB
HALF = 64
L1_CHUNK = 40
L1_NCHUNKS = E_PAD // L1_CHUNK


@functools.lru_cache(maxsize=None)
def _make_seg_sum(edge_split: bool, fd: int, ch: int, nbuf: int):
    nch_tot = E_PAD // ch
    nchunks_w = nch_tot // (NSUB * (2 if edge_split else 1))
    assert nchunks_w % nbuf == 0 and nchunks_w // nbuf >= 3
    NBUF = nbuf
    mesh = plsc.VectorSubcoreMesh(core_axis_name="c", subcore_axis_name="s")

    @functools.partial(
        pl.kernel,
        out_type=jax.ShapeDtypeStruct((NCORES, ACC_ROWS, fd), jnp.float32),
        mesh=mesh,
        scratch_types=[
            pltpu.VMEM_SHARED((ACC_ROWS, fd), jnp.float32),
            pltpu.VMEM((nchunks_w, ch), jnp.int32),
            pltpu.VMEM((nchunks_w, ch), jnp.int32),
            [pltpu.VMEM((ch, fd), jnp.float32)] * NBUF,
            [pltpu.SemaphoreType.DMA] * NBUF,
            [pltpu.SemaphoreType.DMA] * NBUF,
        ],
        compiler_params=pltpu.CompilerParams(use_tc_tiling_on_sc=False),
    )
    def seg_sum(op, src3d, dst2d, zeros, out, acc, src_v, dst_v,
                bufs, gsem, ssem):
        c = lax.axis_index("c")
        s = lax.axis_index("s")
        row_base = pl.multiple_of(s * RPS, 8)
        pltpu.sync_copy(zeros, acc.at[pl.ds(row_base, RPS)])

        if edge_split:
            chunk_base = c * (nch_tot // 2) + s * nchunks_w
        else:
            chunk_base = s * nchunks_w
        chunk_base = pl.multiple_of(chunk_base, 8)
        pltpu.sync_copy(src3d.at[c, pl.ds(chunk_base, nchunks_w)], src_v)
        pltpu.sync_copy(dst2d.at[pl.ds(chunk_base, nchunks_w)], dst_v)
        plsc.subcore_barrier()

        n = nchunks_w
        D = NBUF - 1

        def gather(j, b):
            pltpu.async_copy(op.at[src_v.at[j]], bufs[b], gsem[b])

        def wait_gather(j, b):
            pltpu.make_async_copy(op.at[src_v.at[j]], bufs[b],
                                  gsem[b]).wait()

        def scatter(j, b):
            pltpu.async_copy(bufs[b], acc.at[dst_v.at[j]], ssem[b],
                             add=True)

        def wait_scatter(j, b):
            pltpu.make_async_copy(bufs[b], acc.at[dst_v.at[j]],
                                  ssem[b]).wait()

        for b in range(D):
            gather(b, b)
        for b in range(NBUF):
            wait_gather(b, b)
            scatter(b, b)
            if b == 0:
                gather(D, D)
            else:
                wait_scatter(b - 1, (b + D) % NBUF)
                gather(b + D, (b + D) % NBUF)

        def body(p, carry):
            for b in range(NBUF):
                j = NBUF * p + b
                wait_gather(j, b)
                scatter(j, b)
                wait_scatter(j - 1, (b + D) % NBUF)
                gather(j + D, (b + D) % NBUF)
            return carry

        lax.fori_loop(1, n // NBUF - 1, body, 0)

        for b in range(NBUF):
            j = n - NBUF + b
            if b == 0:
                wait_scatter(j - 1, NBUF - 1)
                gather(j + D, NBUF - 1)
            wait_gather(j, b)
            pltpu.sync_copy(bufs[b], acc.at[dst_v.at[j]], add=True)

        plsc.subcore_barrier()
        pltpu.sync_copy(acc.at[pl.ds(row_base, RPS)],
                        out.at[c, pl.ds(row_base, RPS)])

    return seg_sum


def _seg_sum_l1(*args):
    return _make_seg_sum(edge_split=True, fd=128, ch=L1_CHUNK, nbuf=4)(*args)


def _seg_sum_l2(*args):
    return _make_seg_sum(edge_split=True, fd=64, ch=CHUNK, nbuf=5)(*args)


def _elu(v):
    return jnp.where(v > 0, v, jnp.exp(jnp.minimum(v, 0.0)) - 1.0)


ROWS_BLK = 1000
N_BLKS = N_NODES // ROWS_BLK


def _mlp1_kernel(x_ref, agg_ref, W1_ref, b1_ref, g_ref, be_ref, mu_ref,
                 var_ref, W2_ref, b2_ref, out_ref):
    a = agg_ref[...]
    z = x_ref[...] + a[0] + a[1]
    h = (jnp.dot(z, W1_ref[...], preferred_element_type=jnp.float32)
         + b1_ref[...])
    scale = g_ref[...] * lax.rsqrt(var_ref[...] + 1e-5)
    h = (h - mu_ref[...]) * scale + be_ref[...]
    h = _elu(h)
    h = jnp.dot(h, W2_ref[...], preferred_element_type=jnp.float32) + b2_ref[...]
    out_ref[...] = _elu(h)


def _mlp2_kernel(h_ref, q_ref, b3d_ref, W3_ref, b3_ref, W4_ref, b4_ref,
                 Wr1_ref, br1_ref, Wr2_ref, br2_ref, Wr3_ref, br3_ref,
                 out_ref, g_acc):
    i = pl.program_id(0)
    h = h_ref[...]
    q = q_ref[...]
    z = h + q[0] + q[1]
    t = _elu(jnp.dot(z, W3_ref[...], preferred_element_type=jnp.float32)
             + b3_ref[...])
    p = _elu(jnp.dot(t, W4_ref[...], preferred_element_type=jnp.float32)
             + b4_ref[...])
    bvals = b3d_ref[0]
    onehot_t = (lax.broadcasted_iota(jnp.int32, (N_GRAPHS, ROWS_BLK), 0)
                == bvals).astype(jnp.float32)
    contrib = jnp.dot(onehot_t, p, preferred_element_type=jnp.float32)

    @pl.when(i == 0)
    def _():
        g_acc[...] = contrib

    @pl.when(i > 0)
    def _():
        g_acc[...] = g_acc[...] + contrib

    @pl.when(i == N_BLKS - 1)
    def _():
        g = g_acc[...]
        r = _elu(jnp.dot(g, Wr1_ref[...], preferred_element_type=jnp.float32)
                 + br1_ref[...])
        r = _elu(jnp.dot(r, Wr2_ref[...], preferred_element_type=jnp.float32)
                 + br2_ref[...])
        out_ref[...] = (jnp.dot(r, Wr3_ref[...],
                                preferred_element_type=jnp.float32)
                        + br3_ref[...])


def _full_spec(shape):
    return pl.BlockSpec(shape, lambda i, _s=shape: tuple(0 for _ in _s))


def _mlp1(x, agg, W1, b1, gamma, beta, mean, var, W2, b2):
    return pl.pallas_call(
        _mlp1_kernel,
        grid=(N_BLKS,),
        in_specs=[
            pl.BlockSpec((ROWS_BLK, 128), lambda i: (i, 0)),
            pl.BlockSpec((NCORES, ROWS_BLK, 128), lambda i: (0, i, 0)),
            _full_spec((128, HALF)),
            _full_spec((1, HALF)),
            _full_spec((1, HALF)),
            _full_spec((1, HALF)),
            _full_spec((1, HALF)),
            _full_spec((1, HALF)),
            _full_spec((HALF, HALF)),
            _full_spec((1, HALF)),
        ],
        out_specs=pl.BlockSpec((ROWS_BLK, HALF), lambda i: (i, 0)),
        out_shape=jax.ShapeDtypeStruct((N_NODES, HALF), jnp.float32),
    )(x, agg, W1, b1.reshape(1, -1), gamma.reshape(1, -1),
      beta.reshape(1, -1), mean.reshape(1, -1), var.reshape(1, -1), W2,
      b2.reshape(1, -1))


def _mlp2(h, q, batch3d, W3, b3, W4, b4, Wr1, br1, Wr2, br2, Wr3, br3):
    return pl.pallas_call(
        _mlp2_kernel,
        grid=(N_BLKS,),
        in_specs=[
            pl.BlockSpec((ROWS_BLK, HALF), lambda i: (i, 0)),
            pl.BlockSpec((NCORES, ROWS_BLK, HALF), lambda i: (0, i, 0)),
            pl.BlockSpec((1, 1, ROWS_BLK), lambda i: (i, 0, 0)),
            _full_spec((HALF, HALF)),
            _full_spec((1, HALF)),
            _full_spec((HALF, HALF)),
            _full_spec((1, HALF)),
            _full_spec((HALF, HALF)),
            _full_spec((1, HALF)),
            _full_spec((HALF, 32)),
            _full_spec((1, 32)),
            _full_spec((32, 10)),
            _full_spec((1, 10)),
        ],
        out_specs=_full_spec((N_GRAPHS, 10)),
        out_shape=jax.ShapeDtypeStruct((N_GRAPHS, 10), jnp.float32),
        scratch_shapes=[pltpu.VMEM((N_GRAPHS, N_GRAPHS), jnp.float32)],
    )(h, q, batch3d, W3, b3.reshape(1, -1), W4, b4.reshape(1, -1),
      Wr1, br1.reshape(1, -1), Wr2, br2.reshape(1, -1), Wr3,
      br3.reshape(1, -1))


def kernel(x, edge_index, batch, W1, b1, gamma, beta, mean, var, W2, b2,
           W3, b3, W4, b4, Wr1, br1, Wr2, br2, Wr3, br3):
    src, dst = edge_index[0], edge_index[1]
    pad = E_PAD - N_EDGES
    pad_idx = jnp.arange(pad, dtype=jnp.int32)
    src_p = jnp.concatenate([src, (pad_idx * 97) % N_NODES])
    dst_p = jnp.concatenate([dst, N_NODES + (pad_idx % 16)])
    src_pair = jnp.stack([src_p, src_p])
    src_l1 = src_pair.reshape(NCORES, L1_NCHUNKS, L1_CHUNK)
    dst_l1 = dst_p.reshape(L1_NCHUNKS, L1_CHUNK)
    agg1 = _seg_sum_l1(x, src_l1, dst_l1, jnp.zeros((RPS, 128), jnp.float32))
    h = _mlp1(x, agg1, W1, b1, gamma, beta, mean, var, W2, b2)
    src_l2 = src_pair.reshape(NCORES, NCHUNKS, CHUNK)
    dst_l2 = dst_p.reshape(NCHUNKS, CHUNK)
    agg2 = _seg_sum_l2(h, src_l2, dst_l2,
                       jnp.zeros((RPS, HALF), jnp.float32))
    batch3d = batch.reshape(N_BLKS, 1, ROWS_BLK)
    logits = _mlp2(h, agg2, batch3d, W3, b3, W4, b4, Wr1, br1, Wr2, br2,
                   Wr3, br3)
    return (logits, jnp.zeros((), jnp.float32))


# speedup vs baseline: 1.4762x; 1.0438x over previous
import functools

import jax
import jax.numpy as jnp
from jax import lax
from jax.experimental import pallas as pl
from jax.experimental.pallas import tpu as pltpu
from jax.experimental.pallas import tpu_sc as plsc

N_NODES = 10000
N_EDGES = 320000
N_GRAPHS = 64
CHUNK = 128
NSUB = 16
NCORES = 2
E_PAD = 327680
NCHUNKS = E_PAD // CHUNK
ACC_ROWS = N_NODES + 112
RPS = ACC_ROWS // NSUB
HALF = 64


NBUF = 5


@functools.lru_cache(maxsize=None)
def _make_seg_sum(edge_split: bool):
    nchunks_w = NCHUNKS // (NSUB * (2 if edge_split else 1))
    mesh = plsc.VectorSubcoreMesh(core_axis_name="c", subcore_axis_name="s")

    @functools.partial(
        pl.kernel,
        out_type=jax.ShapeDtypeStruct((NCORES, ACC_ROWS, HALF), jnp.float32),
        mesh=mesh,
        scratch_types=[
            pltpu.VMEM_SHARED((ACC_ROWS, HALF), jnp.float32),
            pltpu.VMEM((nchunks_w, CHUNK), jnp.int32),
            pltpu.VMEM((nchunks_w, CHUNK), jnp.int32),
            [pltpu.VMEM((CHUNK, HALF), jnp.float32)] * NBUF,
            [pltpu.SemaphoreType.DMA] * NBUF,
            [pltpu.SemaphoreType.DMA] * NBUF,
        ],
        compiler_params=pltpu.CompilerParams(use_tc_tiling_on_sc=False),
    )
    def seg_sum(op, src3d, dst2d, zeros, out, acc, src_v, dst_v,
                bufs, gsem, ssem):
        c = lax.axis_index("c")
        s = lax.axis_index("s")
        row_base = pl.multiple_of(s * RPS, 8)
        pltpu.sync_copy(zeros, acc.at[pl.ds(row_base, RPS)])

        if edge_split:
            chunk_base = c * (NCHUNKS // 2) + s * nchunks_w
        else:
            chunk_base = s * nchunks_w
        chunk_base = pl.multiple_of(chunk_base, 8)
        pltpu.sync_copy(src3d.at[c, pl.ds(chunk_base, nchunks_w)], src_v)
        pltpu.sync_copy(dst2d.at[pl.ds(chunk_base, nchunks_w)], dst_v)
        plsc.subcore_barrier()

        n = nchunks_w
        D = NBUF - 1

        def gather(j, b):
            pltpu.async_copy(op.at[src_v.at[j]], bufs[b], gsem[b])

        def wait_gather(j, b):
            pltpu.make_async_copy(op.at[src_v.at[j]], bufs[b],
                                  gsem[b]).wait()

        def scatter(j, b):
            pltpu.async_copy(bufs[b], acc.at[dst_v.at[j]], ssem[b],
                             add=True)

        def wait_scatter(j, b):
            pltpu.make_async_copy(bufs[b], acc.at[dst_v.at[j]],
                                  ssem[b]).wait()

        for b in range(D):
            gather(b, b)
        for b in range(NBUF):
            wait_gather(b, b)
            scatter(b, b)
            if b == 0:
                gather(D, D)
            else:
                wait_scatter(b - 1, (b + D) % NBUF)
                gather(b + D, (b + D) % NBUF)

        def body(p, carry):
            for b in range(NBUF):
                j = NBUF * p + b
                wait_gather(j, b)
                scatter(j, b)
                wait_scatter(j - 1, (b + D) % NBUF)
                gather(j + D, (b + D) % NBUF)
            return carry

        lax.fori_loop(1, n // NBUF - 1, body, 0)

        for b in range(NBUF):
            j = n - NBUF + b
            if b == 0:
                wait_scatter(j - 1, NBUF - 1)
                gather(j + D, NBUF - 1)
            wait_gather(j, b)
            pltpu.sync_copy(bufs[b], acc.at[dst_v.at[j]], add=True)

        plsc.subcore_barrier()
        pltpu.sync_copy(acc.at[pl.ds(row_base, RPS)],
                        out.at[c, pl.ds(row_base, RPS)])

    return seg_sum


def _seg_sum_feat(*args):
    return _make_seg_sum(edge_split=False)(*args)


def _seg_sum_edge(*args):
    return _make_seg_sum(edge_split=True)(*args)


def _elu(v):
    return jnp.where(v > 0, v, jnp.exp(jnp.minimum(v, 0.0)) - 1.0)


ROWS_BLK = 1000
N_BLKS = N_NODES // ROWS_BLK


def _mlp1_kernel(x_ref, agg_ref, W1_ref, b1_ref, g_ref, be_ref, mu_ref,
                 var_ref, W2_ref, b2_ref, out_ref):
    x = x_ref[...]
    a = agg_ref[...]
    z0 = x[:, :HALF] + a[0]
    z1 = x[:, HALF:] + a[1]
    W1 = W1_ref[...]
    h = (jnp.dot(z0, W1[:HALF], preferred_element_type=jnp.float32)
         + jnp.dot(z1, W1[HALF:], preferred_element_type=jnp.float32)
         + b1_ref[...])
    scale = g_ref[...] * lax.rsqrt(var_ref[...] + 1e-5)
    h = (h - mu_ref[...]) * scale + be_ref[...]
    h = _elu(h)
    h = jnp.dot(h, W2_ref[...], preferred_element_type=jnp.float32) + b2_ref[...]
    out_ref[...] = _elu(h)


def _mlp2_kernel(h_ref, q_ref, b3d_ref, W3_ref, b3_ref, W4_ref, b4_ref,
                 Wr1_ref, br1_ref, Wr2_ref, br2_ref, Wr3_ref, br3_ref,
                 out_ref, g_acc):
    i = pl.program_id(0)
    h = h_ref[...]
    q = q_ref[...]
    z = h + q[0] + q[1]
    t = _elu(jnp.dot(z, W3_ref[...], preferred_element_type=jnp.float32)
             + b3_ref[...])
    p = _elu(jnp.dot(t, W4_ref[...], preferred_element_type=jnp.float32)
             + b4_ref[...])
    bvals = b3d_ref[0]
    onehot_t = (lax.broadcasted_iota(jnp.int32, (N_GRAPHS, ROWS_BLK), 0)
                == bvals).astype(jnp.float32)
    contrib = jnp.dot(onehot_t, p, preferred_element_type=jnp.float32)

    @pl.when(i == 0)
    def _():
        g_acc[...] = contrib

    @pl.when(i > 0)
    def _():
        g_acc[...] = g_acc[...] + contrib

    @pl.when(i == N_BLKS - 1)
    def _():
        g = g_acc[...]
        r = _elu(jnp.dot(g, Wr1_ref[...], preferred_element_type=jnp.float32)
                 + br1_ref[...])
        r = _elu(jnp.dot(r, Wr2_ref[...], preferred_element_type=jnp.float32)
                 + br2_ref[...])
        out_ref[...] = (jnp.dot(r, Wr3_ref[...],
                                preferred_element_type=jnp.float32)
                        + br3_ref[...])


def _full_spec(shape):
    return pl.BlockSpec(shape, lambda i, _s=shape: tuple(0 for _ in _s))


def _mlp1(x, agg, W1, b1, gamma, beta, mean, var, W2, b2):
    return pl.pallas_call(
        _mlp1_kernel,
        grid=(N_BLKS,),
        in_specs=[
            pl.BlockSpec((ROWS_BLK, 128), lambda i: (i, 0)),
            pl.BlockSpec((NCORES, ROWS_BLK, HALF), lambda i: (0, i, 0)),
            _full_spec((128, HALF)),
            _full_spec((1, HALF)),
            _full_spec((1, HALF)),
            _full_spec((1, HALF)),
            _full_spec((1, HALF)),
            _full_spec((1, HALF)),
            _full_spec((HALF, HALF)),
            _full_spec((1, HALF)),
        ],
        out_specs=pl.BlockSpec((ROWS_BLK, HALF), lambda i: (i, 0)),
        out_shape=jax.ShapeDtypeStruct((N_NODES, HALF), jnp.float32),
    )(x, agg, W1, b1.reshape(1, -1), gamma.reshape(1, -1),
      beta.reshape(1, -1), mean.reshape(1, -1), var.reshape(1, -1), W2,
      b2.reshape(1, -1))


def _mlp2(h, q, batch3d, W3, b3, W4, b4, Wr1, br1, Wr2, br2, Wr3, br3):
    return pl.pallas_call(
        _mlp2_kernel,
        grid=(N_BLKS,),
        in_specs=[
            pl.BlockSpec((ROWS_BLK, HALF), lambda i: (i, 0)),
            pl.BlockSpec((NCORES, ROWS_BLK, HALF), lambda i: (0, i, 0)),
            pl.BlockSpec((1, 1, ROWS_BLK), lambda i: (i, 0, 0)),
            _full_spec((HALF, HALF)),
            _full_spec((1, HALF)),
            _full_spec((HALF, HALF)),
            _full_spec((1, HALF)),
            _full_spec((HALF, HALF)),
            _full_spec((1, HALF)),
            _full_spec((HALF, 32)),
            _full_spec((1, 32)),
            _full_spec((32, 10)),
            _full_spec((1, 10)),
        ],
        out_specs=_full_spec((N_GRAPHS, 10)),
        out_shape=jax.ShapeDtypeStruct((N_GRAPHS, 10), jnp.float32),
        scratch_shapes=[pltpu.VMEM((N_GRAPHS, N_GRAPHS), jnp.float32)],
    )(h, q, batch3d, W3, b3.reshape(1, -1), W4, b4.reshape(1, -1),
      Wr1, br1.reshape(1, -1), Wr2, br2.reshape(1, -1), Wr3,
      br3.reshape(1, -1))


def kernel(x, edge_index, batch, W1, b1, gamma, beta, mean, var, W2, b2,
           W3, b3, W4, b4, Wr1, br1, Wr2, br2, Wr3, br3):
    src, dst = edge_index[0], edge_index[1]
    pad = E_PAD - N_EDGES
    pad_idx = jnp.arange(pad, dtype=jnp.int32)
    src_p = jnp.concatenate([src, (pad_idx * 97) % N_NODES])
    dst_p = jnp.concatenate([dst, N_NODES + (pad_idx % 16)])
    dst2d = dst_p.reshape(NCHUNKS, CHUNK)
    zeros = jnp.zeros((RPS, HALF), jnp.float32)

    x20 = x.reshape(2 * N_NODES, HALF)
    src_l1 = jnp.stack([2 * src_p, 2 * src_p + 1]).reshape(
        NCORES, NCHUNKS, CHUNK)
    agg1 = _seg_sum_feat(x20, src_l1, dst2d, zeros)
    h = _mlp1(x, agg1, W1, b1, gamma, beta, mean, var, W2, b2)
    src_l2 = jnp.stack([src_p, src_p]).reshape(NCORES, NCHUNKS, CHUNK)
    agg2 = _seg_sum_edge(h, src_l2, dst2d, zeros)
    batch3d = batch.reshape(N_BLKS, 1, ROWS_BLK)
    logits = _mlp2(h, agg2, batch3d, W3, b3, W4, b4, Wr1, br1, Wr2, br2,
                   Wr3, br3)
    return (logits, jnp.zeros((), jnp.float32))


# speedup vs baseline: 1.4825x; 1.0042x over previous
import functools

import jax
import jax.numpy as jnp
from jax import lax
from jax.experimental import pallas as pl
from jax.experimental.pallas import tpu as pltpu
from jax.experimental.pallas import tpu_sc as plsc

N_NODES = 10000
N_EDGES = 320000
N_GRAPHS = 64
CHUNK = 128
NSUB = 16
NCORES = 2
E_PAD = 327680
NCHUNKS = E_PAD // CHUNK
ACC_ROWS = N_NODES + 112
RPS = ACC_ROWS // NSUB
HALF = 64


@functools.lru_cache(maxsize=None)
def _make_seg_sum(edge_split: bool, nbuf: int):
    nchunks_w = NCHUNKS // (NSUB * (2 if edge_split else 1))
    NBUF = nbuf
    assert nchunks_w % NBUF == 0 and nchunks_w // NBUF >= 3
    mesh = plsc.VectorSubcoreMesh(core_axis_name="c", subcore_axis_name="s")

    @functools.partial(
        pl.kernel,
        out_type=jax.ShapeDtypeStruct((NCORES, ACC_ROWS, HALF), jnp.float32),
        mesh=mesh,
        scratch_types=[
            pltpu.VMEM_SHARED((ACC_ROWS, HALF), jnp.float32),
            pltpu.VMEM((nchunks_w, CHUNK), jnp.int32),
            pltpu.VMEM((nchunks_w, CHUNK), jnp.int32),
            [pltpu.VMEM((CHUNK, HALF), jnp.float32)] * NBUF,
            [pltpu.SemaphoreType.DMA] * NBUF,
            [pltpu.SemaphoreType.DMA] * NBUF,
        ],
        compiler_params=pltpu.CompilerParams(use_tc_tiling_on_sc=False),
    )
    def seg_sum(op, src3d, dst2d, zeros, out, acc, src_v, dst_v,
                bufs, gsem, ssem):
        c = lax.axis_index("c")
        s = lax.axis_index("s")
        row_base = pl.multiple_of(s * RPS, 8)
        pltpu.sync_copy(zeros, acc.at[pl.ds(row_base, RPS)])

        if edge_split:
            chunk_base = c * (NCHUNKS // 2) + s * nchunks_w
        else:
            chunk_base = s * nchunks_w
        chunk_base = pl.multiple_of(chunk_base, 8)
        pltpu.sync_copy(src3d.at[c, pl.ds(chunk_base, nchunks_w)], src_v)
        pltpu.sync_copy(dst2d.at[pl.ds(chunk_base, nchunks_w)], dst_v)
        plsc.subcore_barrier()

        n = nchunks_w
        D = NBUF - 1

        def gather(j, b):
            pltpu.async_copy(op.at[src_v.at[j]], bufs[b], gsem[b])

        def wait_gather(j, b):
            pltpu.make_async_copy(op.at[src_v.at[j]], bufs[b],
                                  gsem[b]).wait()

        def scatter(j, b):
            pltpu.async_copy(bufs[b], acc.at[dst_v.at[j]], ssem[b],
                             add=True)

        def wait_scatter(j, b):
            pltpu.make_async_copy(bufs[b], acc.at[dst_v.at[j]],
                                  ssem[b]).wait()

        for b in range(D):
            gather(b, b)
        for b in range(NBUF):
            wait_gather(b, b)
            scatter(b, b)
            if b == 0:
                gather(D, D)
            else:
                wait_scatter(b - 1, (b + D) % NBUF)
                gather(b + D, (b + D) % NBUF)

        def body(p, carry):
            for b in range(NBUF):
                j = NBUF * p + b
                wait_gather(j, b)
                scatter(j, b)
                wait_scatter(j - 1, (b + D) % NBUF)
                gather(j + D, (b + D) % NBUF)
            return carry

        lax.fori_loop(1, n // NBUF - 1, body, 0)

        for b in range(NBUF):
            j = n - NBUF + b
            if b == 0:
                wait_scatter(j - 1, NBUF - 1)
                gather(j + D, NBUF - 1)
            wait_gather(j, b)
            pltpu.sync_copy(bufs[b], acc.at[dst_v.at[j]], add=True)

        plsc.subcore_barrier()
        pltpu.sync_copy(acc.at[pl.ds(row_base, RPS)],
                        out.at[c, pl.ds(row_base, RPS)])

    return seg_sum


def _seg_sum_feat(*args):
    return _make_seg_sum(edge_split=False, nbuf=5)(*args)


def _seg_sum_edge(*args):
    return _make_seg_sum(edge_split=True, nbuf=8)(*args)


def _elu(v):
    return jnp.where(v > 0, v, jnp.exp(jnp.minimum(v, 0.0)) - 1.0)


ROWS_BLK = 1000
N_BLKS = N_NODES // ROWS_BLK


def _mlp1_kernel(x_ref, agg_ref, W1_ref, b1_ref, g_ref, be_ref, mu_ref,
                 var_ref, W2_ref, b2_ref, out_ref):
    x = x_ref[...]
    a = agg_ref[...]
    z0 = x[:, :HALF] + a[0]
    z1 = x[:, HALF:] + a[1]
    W1 = W1_ref[...]
    h = (jnp.dot(z0, W1[:HALF], preferred_element_type=jnp.float32)
         + jnp.dot(z1, W1[HALF:], preferred_element_type=jnp.float32)
         + b1_ref[...])
    scale = g_ref[...] * lax.rsqrt(var_ref[...] + 1e-5)
    h = (h - mu_ref[...]) * scale + be_ref[...]
    h = _elu(h)
    h = jnp.dot(h, W2_ref[...], preferred_element_type=jnp.float32) + b2_ref[...]
    out_ref[...] = _elu(h)


def _mlp2_kernel(h_ref, q_ref, b3d_ref, W3_ref, b3_ref, W4_ref, b4_ref,
                 Wr1_ref, br1_ref, Wr2_ref, br2_ref, Wr3_ref, br3_ref,
                 out_ref, g_acc):
    i = pl.program_id(0)
    h = h_ref[...]
    q = q_ref[...]
    z = h + q[0] + q[1]
    t = _elu(jnp.dot(z, W3_ref[...], preferred_element_type=jnp.float32)
             + b3_ref[...])
    p = _elu(jnp.dot(t, W4_ref[...], preferred_element_type=jnp.float32)
             + b4_ref[...])
    bvals = b3d_ref[0]
    onehot_t = (lax.broadcasted_iota(jnp.int32, (N_GRAPHS, ROWS_BLK), 0)
                == bvals).astype(jnp.float32)
    contrib = jnp.dot(onehot_t, p, preferred_element_type=jnp.float32)

    @pl.when(i == 0)
    def _():
        g_acc[...] = contrib

    @pl.when(i > 0)
    def _():
        g_acc[...] = g_acc[...] + contrib

    @pl.when(i == N_BLKS - 1)
    def _():
        g = g_acc[...]
        r = _elu(jnp.dot(g, Wr1_ref[...], preferred_element_type=jnp.float32)
                 + br1_ref[...])
        r = _elu(jnp.dot(r, Wr2_ref[...], preferred_element_type=jnp.float32)
                 + br2_ref[...])
        out_ref[...] = (jnp.dot(r, Wr3_ref[...],
                                preferred_element_type=jnp.float32)
                        + br3_ref[...])


def _full_spec(shape):
    return pl.BlockSpec(shape, lambda i, _s=shape: tuple(0 for _ in _s))


def _mlp1(x, agg, W1, b1, gamma, beta, mean, var, W2, b2):
    return pl.pallas_call(
        _mlp1_kernel,
        grid=(N_BLKS,),
        in_specs=[
            pl.BlockSpec((ROWS_BLK, 128), lambda i: (i, 0)),
            pl.BlockSpec((NCORES, ROWS_BLK, HALF), lambda i: (0, i, 0)),
            _full_spec((128, HALF)),
            _full_spec((1, HALF)),
            _full_spec((1, HALF)),
            _full_spec((1, HALF)),
            _full_spec((1, HALF)),
            _full_spec((1, HALF)),
            _full_spec((HALF, HALF)),
            _full_spec((1, HALF)),
        ],
        out_specs=pl.BlockSpec((ROWS_BLK, HALF), lambda i: (i, 0)),
        out_shape=jax.ShapeDtypeStruct((N_NODES, HALF), jnp.float32),
    )(x, agg, W1, b1.reshape(1, -1), gamma.reshape(1, -1),
      beta.reshape(1, -1), mean.reshape(1, -1), var.reshape(1, -1), W2,
      b2.reshape(1, -1))


def _mlp2(h, q, batch3d, W3, b3, W4, b4, Wr1, br1, Wr2, br2, Wr3, br3):
    return pl.pallas_call(
        _mlp2_kernel,
        grid=(N_BLKS,),
        in_specs=[
            pl.BlockSpec((ROWS_BLK, HALF), lambda i: (i, 0)),
            pl.BlockSpec((NCORES, ROWS_BLK, HALF), lambda i: (0, i, 0)),
            pl.BlockSpec((1, 1, ROWS_BLK), lambda i: (i, 0, 0)),
            _full_spec((HALF, HALF)),
            _full_spec((1, HALF)),
            _full_spec((HALF, HALF)),
            _full_spec((1, HALF)),
            _full_spec((HALF, HALF)),
            _full_spec((1, HALF)),
            _full_spec((HALF, 32)),
            _full_spec((1, 32)),
            _full_spec((32, 10)),
            _full_spec((1, 10)),
        ],
        out_specs=_full_spec((N_GRAPHS, 10)),
        out_shape=jax.ShapeDtypeStruct((N_GRAPHS, 10), jnp.float32),
        scratch_shapes=[pltpu.VMEM((N_GRAPHS, N_GRAPHS), jnp.float32)],
    )(h, q, batch3d, W3, b3.reshape(1, -1), W4, b4.reshape(1, -1),
      Wr1, br1.reshape(1, -1), Wr2, br2.reshape(1, -1), Wr3,
      br3.reshape(1, -1))


def kernel(x, edge_index, batch, W1, b1, gamma, beta, mean, var, W2, b2,
           W3, b3, W4, b4, Wr1, br1, Wr2, br2, Wr3, br3):
    src, dst = edge_index[0], edge_index[1]
    pad = E_PAD - N_EDGES
    pad_idx = jnp.arange(pad, dtype=jnp.int32)
    src_p = jnp.concatenate([src, (pad_idx * 97) % N_NODES])
    dst_p = jnp.concatenate([dst, N_NODES + (pad_idx % 16)])
    dst2d = dst_p.reshape(NCHUNKS, CHUNK)
    zeros = jnp.zeros((RPS, HALF), jnp.float32)

    x20 = x.reshape(2 * N_NODES, HALF)
    src_l1 = jnp.stack([2 * src_p, 2 * src_p + 1]).reshape(
        NCORES, NCHUNKS, CHUNK)
    agg1 = _seg_sum_feat(x20, src_l1, dst2d, zeros)
    h = _mlp1(x, agg1, W1, b1, gamma, beta, mean, var, W2, b2)
    src_l2 = jnp.stack([src_p, src_p]).reshape(NCORES, NCHUNKS, CHUNK)
    agg2 = _seg_sum_edge(h, src_l2, dst2d, zeros)
    batch3d = batch.reshape(N_BLKS, 1, ROWS_BLK)
    logits = _mlp2(h, agg2, batch3d, W3, b3, W4, b4, Wr1, br1, Wr2, br2,
                   Wr3, br3)
    return (logits, jnp.zeros((), jnp.float32))
